# baseline (speedup 1.0000x reference)
import functools

import jax
import jax.numpy as jnp
from jax import lax
from jax.experimental import pallas as pl
from jax.experimental.pallas import tpu as pltpu
from jax.experimental.pallas import tpu_sc as plsc

D_EMB = 64
LANES = 16
NW = 32


def _wid():
    return lax.axis_index("s") * 2 + lax.axis_index("c")


def _make_compact(vocab):
    n_full = vocab // 128
    tail = vocab - n_full * 128
    per_tile = (n_full + NW - 1) // NW
    n_pairs = (per_tile + 1) // 2
    mesh = plsc.VectorSubcoreMesh(core_axis_name="c", subcore_axis_name="s")

    @functools.partial(
        pl.kernel,
        out_type=jax.ShapeDtypeStruct((vocab // 2, 2 * D_EMB), jnp.float32),
        mesh=mesh,
        compiler_params=pltpu.CompilerParams(needs_layout_passes=False),
        scratch_types=[
            pltpu.VMEM((2, D_EMB, 128), jnp.float32),
            pltpu.VMEM((2, D_EMB, 128), jnp.float32),
            pltpu.SemaphoreType.DMA,
            pltpu.SemaphoreType.DMA,
            pltpu.SemaphoreType.DMA,
            pltpu.SemaphoreType.DMA,
        ],
    )
    def k(wt_t, tail_c, wtc, inb, outb, isem0, isem1, osem0, osem1):
        wid = _wid()
        isems = (isem0, isem1)
        osems = (osem0, osem1)

        def blk(i):
            return wid + i * NW

        def fire_in(i, slot):
            @pl.when(blk(i) < n_full)
            def _():
                pltpu.async_copy(
                    wt_t.at[:, pl.ds(blk(i) * 128, 128)], inb.at[slot],
                    isems[slot])

        def drain_in(i, slot):
            @pl.when(blk(i) < n_full)
            def _():
                pltpu.make_async_copy(
                    wt_t.at[:, pl.ds(0, 128)], inb.at[slot], isems[slot]).wait()

        def drain_out(i, slot):
            @pl.when(blk(i) < n_full)
            def _():
                pltpu.make_async_copy(
                    wt_t.at[:, pl.ds(0, 128)], outb.at[slot], osems[slot]).wait()

        rows_c = [(jg * LANES + lax.iota(jnp.int32, LANES)) >> 1
                  for jg in range(8)]
        cols_c = [((jg * LANES + lax.iota(jnp.int32, LANES)) & 1) * D_EMB
                  for jg in range(8)]
        one = jnp.full((LANES,), 1, jnp.int32)

        def transpose(i, slot):
            @pl.when(blk(i) < n_full)
            def _():
                def drow(d, cols):
                    new = []
                    for jg in range(8):
                        v = inb[slot, d, pl.ds(jg * LANES, LANES)]
                        plsc.store_scatter(outb.at[slot], [rows_c[jg], cols[jg]], v)
                        new.append(cols[jg] + one)
                    return tuple(new)

                lax.fori_loop(0, D_EMB, drow, tuple(cols_c))

        def fire_out(i, slot):
            @pl.when(blk(i) < n_full)
            def _():
                pltpu.async_copy(
                    outb.at[slot], wtc.at[pl.ds(blk(i) * D_EMB, D_EMB)],
                    osems[slot])

        fire_in(0, 0)

        def body(p, carry):
            a = 2 * p
            drain_in(a, 0)
            fire_in(a + 1, 1)

            @pl.when(p > 0)
            def _():
                drain_out(a - 2, 0)
                drain_out(a - 1, 1)

            transpose(a, 0)
            fire_out(a, 0)
            drain_in(a + 1, 1)
            fire_in(a + 2, 0)
            transpose(a + 1, 1)
            fire_out(a + 1, 1)
            return carry

        lax.fori_loop(0, n_pairs, body, 0)
        drain_out(2 * n_pairs - 2, 0)
        drain_out(2 * n_pairs - 1, 1)

        if tail:
            @pl.when(wid == NW - 1)
            def _():
                pltpu.async_copy(
                    tail_c, outb.at[0, pl.ds(0, tail // 2)], isem0).wait()
                pltpu.async_copy(
                    outb.at[0, pl.ds(0, tail // 2)],
                    wtc.at[pl.ds(n_full * D_EMB, tail // 2)], isem0).wait()

    return k


def _make_gather(batch, hist, vocab):
    mesh = plsc.VectorSubcoreMesh(core_axis_name="c", subcore_axis_name="s")
    n_pairs = hist // 2

    @functools.partial(
        pl.kernel,
        out_type=jax.ShapeDtypeStruct((hist, D_EMB, batch), jnp.float32),
        mesh=mesh,
        compiler_params=pltpu.CompilerParams(needs_layout_passes=False),
        scratch_types=[
            pltpu.VMEM((hist, 128), jnp.int32),
            pltpu.VMEM((2, 128), jnp.int32),
            pltpu.VMEM((2, 128, 128), jnp.float32),
            pltpu.VMEM((2, D_EMB, 128), jnp.float32),
            pltpu.SemaphoreType.DMA,
            pltpu.SemaphoreType.DMA,
            pltpu.SemaphoreType.DMA,
            pltpu.SemaphoreType.DMA,
        ],
    )
    def k(idx_t, wtc, out, idx_v, qb, rows, outb, gsem0, gsem1, osem0, osem1):
        wid = _wid()
        b0 = wid * 128
        gsems = (gsem0, gsem1)
        osems = (osem0, osem1)
        pltpu.async_copy(idx_t.at[:, pl.ds(b0, 128)], idx_v, gsem0).wait()

        rows_c = [jg * LANES + lax.iota(jnp.int32, LANES) for jg in range(8)]
        one = jnp.full((LANES,), 1, jnp.int32)

        def fire_gather(h, slot):
            for jg in range(8):
                t = idx_v[h, pl.ds(jg * LANES, LANES)]
                qb[slot, pl.ds(jg * LANES, LANES)] = t >> 1
            pltpu.async_copy(wtc.at[qb.at[slot]], rows.at[slot], gsems[slot])

        def drain_gather(slot):
            pltpu.make_async_copy(
                wtc.at[pl.ds(0, 128)], rows.at[slot], gsems[slot]).wait()

        def drain_out(slot):
            pltpu.make_async_copy(
                wtc.at[pl.ds(0, D_EMB)], outb.at[slot], osems[slot]).wait()

        def repack(h, slot):
            cols0 = []
            for jg in range(8):
                t = idx_v[h, pl.ds(jg * LANES, LANES)]
                cols0.append((t & 1) * D_EMB)

            def drow(d, cols):
                new = []
                for jg in range(8):
                    v = plsc.load_gather(rows.at[slot], [rows_c[jg], cols[jg]])
                    outb[slot, d, pl.ds(jg * LANES, LANES)] = v
                    new.append(cols[jg] + one)
                return tuple(new)

            lax.fori_loop(0, D_EMB, drow, tuple(cols0))

        def fire_out(h, slot):
            pltpu.async_copy(
                outb.at[slot], out.at[h, :, pl.ds(b0, 128)], osems[slot])

        fire_gather(0, 0)

        def body(p, carry):
            h = 2 * p
            fire_gather(h + 1, 1)
            drain_gather(0)

            @pl.when(p > 0)
            def _():
                drain_out(0)

            repack(h, 0)
            fire_out(h, 0)

            @pl.when(p < n_pairs - 1)
            def _():
                fire_gather(h + 2, 0)

            drain_gather(1)

            @pl.when(p > 0)
            def _():
                drain_out(1)

            repack(h + 1, 1)
            fire_out(h + 1, 1)
            return carry

        lax.fori_loop(0, n_pairs, body, 0)
        drain_out(0)
        drain_out(1)

    return k


@functools.partial(jax.jit, static_argnums=(2, 3, 4))
def _embedding(indices, weight, batch, hist, vocab):
    n_full = vocab // 128
    tail = vocab - n_full * 128
    if tail:
        tail_c = weight[n_full * 128:].reshape(tail // 2, 2 * D_EMB)
    else:
        tail_c = jnp.zeros((0, 2 * D_EMB), jnp.float32)
    wtc = _make_compact(vocab)(weight.T, tail_c)
    out = _make_gather(batch, hist, vocab)(indices.T, wtc)
    return out.transpose(2, 0, 1)


def kernel(indices, weight):
    b, h = indices.shape
    v, d = weight.shape
    assert d == D_EMB
    return _embedding(indices, weight, b, h, v)


# speedup vs baseline: 2.4415x; 2.4415x over previous
import functools

import jax
import jax.numpy as jnp
from jax import lax
from jax.experimental import pallas as pl
from jax.experimental.pallas import tpu as pltpu
from jax.experimental.pallas import tpu_sc as plsc

D_EMB = 64
LANES = 16
NW = 32


def _wid():
    return lax.axis_index("s") * 2 + lax.axis_index("c")


def _make_spread(vocab):
    n_full = vocab // 128
    tail = vocab - n_full * 128
    per_tile = (n_full + NW - 1) // NW
    n_pairs = (per_tile + 1) // 2
    mesh = plsc.VectorSubcoreMesh(core_axis_name="c", subcore_axis_name="s")

    @functools.partial(
        pl.kernel,
        out_type=jax.ShapeDtypeStruct((vocab, 2 * D_EMB), jnp.float32),
        mesh=mesh,
        compiler_params=pltpu.CompilerParams(needs_layout_passes=False),
        scratch_types=[
            pltpu.VMEM((2, D_EMB, 128), jnp.float32),
            pltpu.VMEM((2, 128, 128), jnp.float32),
            pltpu.VMEM((32, 128), jnp.float32),
            pltpu.SemaphoreType.DMA,
            pltpu.SemaphoreType.DMA,
            pltpu.SemaphoreType.DMA,
            pltpu.SemaphoreType.DMA,
        ],
    )
    def k(wt_t, tail_c, wtr, inb, outb, tstage, isem0, isem1, osem0, osem1):
        wid = _wid()
        isems = (isem0, isem1)
        osems = (osem0, osem1)

        rows_c = [jg * LANES + lax.iota(jnp.int32, LANES) for jg in range(8)]
        iota = lax.iota(jnp.int32, LANES)
        one = jnp.full((LANES,), 1, jnp.int32)

        def blk(i):
            return wid + i * NW

        def fire_in(i, slot):
            @pl.when(blk(i) < n_full)
            def _():
                pltpu.async_copy(
                    wt_t.at[:, pl.ds(blk(i) * 128, 128)], inb.at[slot],
                    isems[slot])

        def drain_in(i, slot):
            @pl.when(blk(i) < n_full)
            def _():
                pltpu.make_async_copy(
                    wt_t.at[:, pl.ds(0, 128)], inb.at[slot], isems[slot]).wait()

        def drain_out(i, slot):
            @pl.when(blk(i) < n_full)
            def _():
                pltpu.make_async_copy(
                    wtr.at[pl.ds(0, 128)], outb.at[slot], osems[slot]).wait()

        def transpose(i, slot):
            @pl.when(blk(i) < n_full)
            def _():
                def drow(d, cols):
                    for jg in range(8):
                        v = inb[slot, d, pl.ds(jg * LANES, LANES)]
                        plsc.store_scatter(outb.at[slot], [rows_c[jg], cols], v)
                    return cols + one

                lax.fori_loop(0, D_EMB, drow, iota)

        def fire_out(i, slot):
            @pl.when(blk(i) < n_full)
            def _():
                pltpu.async_copy(
                    outb.at[slot], wtr.at[pl.ds(blk(i) * 128, 128)],
                    osems[slot])

        fire_in(0, 0)

        def body(p, carry):
            a = 2 * p
            drain_in(a, 0)
            fire_in(a + 1, 1)

            @pl.when(p > 0)
            def _():
                drain_out(a - 2, 0)
                drain_out(a - 1, 1)

            transpose(a, 0)
            fire_out(a, 0)
            drain_in(a + 1, 1)
            fire_in(a + 2, 0)
            transpose(a + 1, 1)
            fire_out(a + 1, 1)
            return carry

        lax.fori_loop(0, n_pairs, body, 0)
        drain_out(2 * n_pairs - 2, 0)
        drain_out(2 * n_pairs - 1, 1)

        if tail:
            @pl.when(wid == NW - 1)
            def _():
                pltpu.async_copy(tail_c, tstage, isem0).wait()

                def drow(d, carry2):
                    cols_dst, cols_src = carry2
                    for jg in range(tail // LANES):
                        j16 = jg * LANES + iota
                        v = plsc.load_gather(
                            tstage, [j16 >> 1, cols_src + (j16 & 1) * D_EMB])
                        plsc.store_scatter(
                            outb.at[0], [j16, cols_dst], v)
                    return (cols_dst + one, cols_src + one)

                lax.fori_loop(0, D_EMB, drow, (iota & 15, iota * 0))
                pltpu.async_copy(
                    outb.at[0, pl.ds(0, tail)],
                    wtr.at[pl.ds(n_full * 128, tail)], isem0).wait()

    return k


def _make_gather(batch, hist, vocab):
    mesh = plsc.VectorSubcoreMesh(core_axis_name="c", subcore_axis_name="s")
    n_pairs = hist // 2

    @functools.partial(
        pl.kernel,
        out_type=jax.ShapeDtypeStruct((hist, D_EMB, batch), jnp.float32),
        mesh=mesh,
        compiler_params=pltpu.CompilerParams(needs_layout_passes=False),
        scratch_types=[
            pltpu.VMEM((hist, 128), jnp.int32),
            pltpu.VMEM((2, 128, 128), jnp.float32),
            pltpu.VMEM((2, D_EMB, 128), jnp.float32),
            pltpu.SemaphoreType.DMA,
            pltpu.SemaphoreType.DMA,
            pltpu.SemaphoreType.DMA,
            pltpu.SemaphoreType.DMA,
        ],
    )
    def k(idx_t, wtr, out, idx_v, rows, outb, gsem0, gsem1, osem0, osem1):
        wid = _wid()
        b0 = wid * 128
        gsems = (gsem0, gsem1)
        osems = (osem0, osem1)
        pltpu.async_copy(idx_t.at[:, pl.ds(b0, 128)], idx_v, gsem0).wait()

        rows_c = [jg * LANES + lax.iota(jnp.int32, LANES) for jg in range(8)]
        one = jnp.full((LANES,), 1, jnp.int32)

        def fire_gather(h, slot):
            pltpu.async_copy(wtr.at[idx_v.at[h]], rows.at[slot], gsems[slot])

        def drain_gather(slot):
            pltpu.make_async_copy(
                wtr.at[pl.ds(0, 128)], rows.at[slot], gsems[slot]).wait()

        def drain_out(slot):
            pltpu.make_async_copy(
                wtr.at[pl.ds(0, D_EMB)], outb.at[slot], osems[slot]).wait()

        def repack(h, slot):
            cols0 = [(idx_v[h, pl.ds(jg * LANES, LANES)] & 15)
                     for jg in range(8)]

            def drow(d, cols):
                new = []
                for jg in range(8):
                    v = plsc.load_gather(rows.at[slot], [rows_c[jg], cols[jg]])
                    outb[slot, d, pl.ds(jg * LANES, LANES)] = v
                    new.append(cols[jg] + one)
                return tuple(new)

            lax.fori_loop(0, D_EMB, drow, tuple(cols0))

        def fire_out(h, slot):
            pltpu.async_copy(
                outb.at[slot], out.at[h, :, pl.ds(b0, 128)], osems[slot])

        fire_gather(0, 0)

        def body(p, carry):
            h = 2 * p
            fire_gather(h + 1, 1)
            drain_gather(0)

            @pl.when(p > 0)
            def _():
                drain_out(0)

            repack(h, 0)
            fire_out(h, 0)

            @pl.when(p < n_pairs - 1)
            def _():
                fire_gather(h + 2, 0)

            drain_gather(1)

            @pl.when(p > 0)
            def _():
                drain_out(1)

            repack(h + 1, 1)
            fire_out(h + 1, 1)
            return carry

        lax.fori_loop(0, n_pairs, body, 0)
        drain_out(0)
        drain_out(1)

    return k


@functools.partial(jax.jit, static_argnums=(2, 3, 4))
def _embedding(indices, weight, batch, hist, vocab):
    n_full = vocab // 128
    tail = vocab - n_full * 128
    if tail:
        tail_c = weight[n_full * 128:].reshape(tail // 2, 2 * D_EMB)
    else:
        tail_c = jnp.zeros((0, 2 * D_EMB), jnp.float32)
    wtr = _make_spread(vocab)(weight.T, tail_c)
    out = _make_gather(batch, hist, vocab)(indices.T, wtr)
    return out.transpose(2, 0, 1)


def kernel(indices, weight):
    b, h = indices.shape
    v, d = weight.shape
    assert d == D_EMB
    return _embedding(indices, weight, b, h, v)


# speedup vs baseline: 2.4809x; 1.0161x over previous
import functools

import jax
import jax.numpy as jnp
from jax import lax
from jax.experimental import pallas as pl
from jax.experimental.pallas import tpu as pltpu
from jax.experimental.pallas import tpu_sc as plsc

D_EMB = 64
LANES = 16
NW = 32
SB = 256


def _wid():
    return lax.axis_index("s") * 2 + lax.axis_index("c")


def _make_spread(vocab):
    n_full = vocab // SB
    tail = vocab - n_full * SB
    per_tile = (n_full + NW - 1) // NW
    n_pairs = (per_tile + 1) // 2
    mesh = plsc.VectorSubcoreMesh(core_axis_name="c", subcore_axis_name="s")

    @functools.partial(
        pl.kernel,
        out_type=jax.ShapeDtypeStruct((vocab, 2 * D_EMB), jnp.float32),
        mesh=mesh,
        compiler_params=pltpu.CompilerParams(needs_layout_passes=False),
        scratch_types=[
            pltpu.VMEM((2, D_EMB, SB), jnp.float32),
            pltpu.VMEM((2, SB, 128), jnp.float32),
            pltpu.VMEM((32, 128), jnp.float32),
            pltpu.SemaphoreType.DMA,
            pltpu.SemaphoreType.DMA,
            pltpu.SemaphoreType.DMA,
            pltpu.SemaphoreType.DMA,
        ],
    )
    def k(wt_t, tail_c, wtr, inb, outb, tstage, isem0, isem1, osem0, osem1):
        wid = _wid()
        isems = (isem0, isem1)
        osems = (osem0, osem1)

        n_jg = SB // LANES
        rows_c = [jg * LANES + lax.iota(jnp.int32, LANES) for jg in range(n_jg)]
        iota = lax.iota(jnp.int32, LANES)
        one = jnp.full((LANES,), 1, jnp.int32)

        def blk(i):
            return wid + i * NW

        def fire_in(i, slot):
            @pl.when(blk(i) < n_full)
            def _():
                pltpu.async_copy(
                    wt_t.at[:, pl.ds(blk(i) * SB, SB)], inb.at[slot],
                    isems[slot])

        def drain_in(i, slot):
            @pl.when(blk(i) < n_full)
            def _():
                pltpu.make_async_copy(
                    wt_t.at[:, pl.ds(0, SB)], inb.at[slot], isems[slot]).wait()

        def drain_out(i, slot):
            @pl.when(blk(i) < n_full)
            def _():
                pltpu.make_async_copy(
                    wtr.at[pl.ds(0, SB)], outb.at[slot], osems[slot]).wait()

        def transpose(i, slot):
            @pl.when(blk(i) < n_full)
            def _():
                def drow(d, cols):
                    for jg in range(n_jg):
                        v = inb[slot, d, pl.ds(jg * LANES, LANES)]
                        plsc.store_scatter(outb.at[slot], [rows_c[jg], cols], v)
                    return cols + one

                lax.fori_loop(0, D_EMB, drow, iota)

        def fire_out(i, slot):
            @pl.when(blk(i) < n_full)
            def _():
                pltpu.async_copy(
                    outb.at[slot], wtr.at[pl.ds(blk(i) * SB, SB)],
                    osems[slot])

        fire_in(0, 0)

        def body(p, carry):
            a = 2 * p
            drain_in(a, 0)
            fire_in(a + 1, 1)

            @pl.when(p > 0)
            def _():
                drain_out(a - 2, 0)
                drain_out(a - 1, 1)

            transpose(a, 0)
            fire_out(a, 0)
            drain_in(a + 1, 1)
            fire_in(a + 2, 0)
            transpose(a + 1, 1)
            fire_out(a + 1, 1)
            return carry

        lax.fori_loop(0, n_pairs, body, 0)
        drain_out(2 * n_pairs - 2, 0)
        drain_out(2 * n_pairs - 1, 1)

        if tail:
            @pl.when(wid == NW - 1)
            def _():
                pltpu.async_copy(tail_c, tstage, isem0).wait()

                def drow(d, carry2):
                    cols_dst, cols_src = carry2
                    for jg in range(tail // LANES):
                        j16 = jg * LANES + iota
                        v = plsc.load_gather(
                            tstage, [j16 >> 1, cols_src + (j16 & 1) * D_EMB])
                        plsc.store_scatter(
                            outb.at[0], [j16, cols_dst], v)
                    return (cols_dst + one, cols_src + one)

                lax.fori_loop(0, D_EMB, drow, (iota & 15, iota * 0))
                pltpu.async_copy(
                    outb.at[0, pl.ds(0, tail)],
                    wtr.at[pl.ds(n_full * SB, tail)], isem0).wait()

    return k


def _make_gather(batch, hist, vocab):
    mesh = plsc.VectorSubcoreMesh(core_axis_name="c", subcore_axis_name="s")
    n_rounds = hist // 4

    @functools.partial(
        pl.kernel,
        out_type=jax.ShapeDtypeStruct((hist, D_EMB, batch), jnp.float32),
        mesh=mesh,
        compiler_params=pltpu.CompilerParams(needs_layout_passes=False),
        scratch_types=[
            pltpu.VMEM((hist, 128), jnp.int32),
            pltpu.VMEM((2, 2, 128, 128), jnp.float32),
            pltpu.VMEM((2, 2, D_EMB, 128), jnp.float32),
            pltpu.SemaphoreType.DMA,
            pltpu.SemaphoreType.DMA,
            pltpu.SemaphoreType.DMA,
            pltpu.SemaphoreType.DMA,
        ],
    )
    def k(idx_t, wtr, out, idx_v, rows, outb, gsem0, gsem1, osem0, osem1):
        wid = _wid()
        b0 = wid * 128
        gsems = (gsem0, gsem1)
        osems = (osem0, osem1)
        pltpu.async_copy(idx_t.at[:, pl.ds(b0, 128)], idx_v, gsem0).wait()

        rows_c = [jg * LANES + lax.iota(jnp.int32, LANES) for jg in range(8)]
        one = jnp.full((LANES,), 1, jnp.int32)

        def fire_pair(a, slot):
            for kk in range(2):
                pltpu.async_copy(
                    wtr.at[idx_v.at[2 * a + kk]], rows.at[slot, kk],
                    gsems[slot])

        def drain_pair(slot):
            for _ in range(2):
                pltpu.make_async_copy(
                    wtr.at[pl.ds(0, 128)], rows.at[slot, 0], gsems[slot]).wait()

        def drain_outs(slot):
            for _ in range(2):
                pltpu.make_async_copy(
                    wtr.at[pl.ds(0, D_EMB)], outb.at[slot, 0],
                    osems[slot]).wait()

        def repack(h, slot, kk):
            cols0 = [(idx_v[h, pl.ds(jg * LANES, LANES)] & 15)
                     for jg in range(8)]

            def drow(d, cols):
                new = []
                for jg in range(8):
                    v = plsc.load_gather(
                        rows.at[slot, kk], [rows_c[jg], cols[jg]])
                    outb[slot, kk, d, pl.ds(jg * LANES, LANES)] = v
                    new.append(cols[jg] + one)
                return tuple(new)

            lax.fori_loop(0, D_EMB, drow, tuple(cols0))

        def do_pair(a, slot):
            for kk in range(2):
                h = 2 * a + kk
                repack(h, slot, kk)
                pltpu.async_copy(
                    outb.at[slot, kk], out.at[h, :, pl.ds(b0, 128)],
                    osems[slot])

        fire_pair(0, 0)

        def body(p, carry):
            a = 2 * p
            fire_pair(a + 1, 1)
            drain_pair(0)

            @pl.when(p > 0)
            def _():
                drain_outs(0)

            do_pair(a, 0)

            @pl.when(p < n_rounds - 1)
            def _():
                fire_pair(a + 2, 0)

            drain_pair(1)

            @pl.when(p > 0)
            def _():
                drain_outs(1)

            do_pair(a + 1, 1)
            return carry

        lax.fori_loop(0, n_rounds, body, 0)
        drain_outs(0)
        drain_outs(1)

    return k


@functools.partial(jax.jit, static_argnums=(2, 3, 4))
def _embedding(indices, weight, batch, hist, vocab):
    n_full = vocab // SB
    tail = vocab - n_full * SB
    if tail:
        tail_c = weight[n_full * SB:].reshape(tail // 2, 2 * D_EMB)
    else:
        tail_c = jnp.zeros((0, 2 * D_EMB), jnp.float32)
    wtr = _make_spread(vocab)(weight.T, tail_c)
    out = _make_gather(batch, hist, vocab)(indices.T, wtr)
    return out.transpose(2, 0, 1)


def kernel(indices, weight):
    b, h = indices.shape
    v, d = weight.shape
    assert d == D_EMB
    return _embedding(indices, weight, b, h, v)
